# baseline (speedup 1.0000x reference)
import jax
import jax.numpy as jnp
from jax.experimental import pallas as pl
from jax.experimental.pallas import tpu as pltpu


H = W = 16
C0, C1, C2 = 3, 8, 16
HP, WP = H // 2, W // 2
_NB = 256


def _conv_block(a, w_ref, b_ref, relu, shift=1):
    nb, rows, kin = a.shape
    kout = w_ref.shape[1] // 3
    z = jnp.dot(a.reshape(nb * rows, kin), w_ref[...],
                preferred_element_type=jnp.float32)
    z = z.reshape(nb, rows, 3 * kout)
    z0 = z[..., :kout]
    z1 = z[..., kout:2 * kout]
    z2 = z[..., 2 * kout:]
    zero = jnp.zeros((nb, shift, kout), jnp.float32)
    acc = (z1
           + jnp.concatenate([zero, z0[:, :rows - shift]], axis=1)
           + jnp.concatenate([z2[:, shift:], zero], axis=1)
           + b_ref[...])
    if relu:
        acc = jnp.maximum(acc, 0.0)
    return acc


def _fwd_kernel(x_ref, mean_ref, w1_ref, b1_ref, w2_ref, b2_ref,
                w3_ref, b3_ref, o_ref):
    nb = x_ref.shape[0]
    x = x_ref[...]

    lhs = jnp.concatenate([x[:, c] for c in range(C0)], axis=-1)
    lhs = (lhs - mean_ref[...]).astype(jnp.bfloat16)

    a1 = _conv_block(lhs, w1_ref, b1_ref, relu=True).astype(jnp.bfloat16)
    a2 = _conv_block(a1, w2_ref, b2_ref, relu=True).astype(jnp.bfloat16)

    a2_dn = jnp.concatenate([a2[:, 1:], a2[:, :1]], axis=1)
    m = jnp.maximum(a2, a2_dn)
    m_shift = jnp.concatenate([m[..., 1:], m[..., :1]], axis=-1)
    mw = jnp.maximum(m, m_shift)

    a3 = _conv_block(mw, w3_ref, b3_ref, relu=False, shift=2)
    o_ref[...] = a3.astype(o_ref.dtype)


def kernel(x_nchw, mean_l, t1, b1, t2, b2, t3, b3):
    N = x_nchw.shape[0]
    nb = min(_NB, N)
    nblocks = pl.cdiv(N, nb)
    npad = nblocks * nb
    if npad != N:
        x_nchw = jnp.pad(x_nchw, ((0, npad - N), (0, 0), (0, 0), (0, 0)))

    w1 = jnp.concatenate([t1[0], t1[1], t1[2]], axis=1).astype(jnp.bfloat16)
    w2 = jnp.concatenate([t2[0], t2[1], t2[2]], axis=1).astype(jnp.bfloat16)
    w3 = jnp.concatenate([t3[0], t3[1], t3[2]], axis=1).astype(jnp.bfloat16)

    out = pl.pallas_call(
        _fwd_kernel,
        out_shape=jax.ShapeDtypeStruct((npad, H, C2 * WP), x_nchw.dtype),
        grid=(nblocks,),
        in_specs=[
            pl.BlockSpec((nb, C0, H, W), lambda n: (n, 0, 0, 0)),
            pl.BlockSpec((1, 1, C0 * W), lambda n: (0, 0, 0)),
            pl.BlockSpec((C0 * W, 3 * C1 * W), lambda n: (0, 0)),
            pl.BlockSpec((1, C1 * W), lambda n: (0, 0)),
            pl.BlockSpec((C1 * W, 3 * C1 * W), lambda n: (0, 0)),
            pl.BlockSpec((1, C1 * W), lambda n: (0, 0)),
            pl.BlockSpec((C1 * W, 3 * C2 * WP), lambda n: (0, 0)),
            pl.BlockSpec((1, C2 * WP), lambda n: (0, 0)),
        ],
        out_specs=pl.BlockSpec((nb, H, C2 * WP), lambda n: (n, 0, 0)),
        compiler_params=pltpu.CompilerParams(
            dimension_semantics=("parallel",)),
    )(x_nchw, mean_l, w1, b1, w2, b2, w3, b3)

    out = out[:N, ::2]
    return jnp.transpose(out.reshape(N, HP, C2, WP), (0, 2, 1, 3))


# speedup vs baseline: 1.3532x; 1.3532x over previous
import jax
import jax.numpy as jnp
from jax.experimental import pallas as pl
from jax.experimental.pallas import tpu as pltpu


H = W = 16
C0, C1, C2 = 3, 8, 16
HP, WP = H // 2, W // 2
_NB = 256


def _conv_block(a, w_ref, b_ref, relu, shift=1):
    nb, rows, kin = a.shape
    kout = w_ref.shape[1] // 3
    z = jnp.dot(a.reshape(nb * rows, kin), w_ref[...],
                preferred_element_type=jnp.float32)
    z = z.reshape(nb, rows, 3 * kout)
    z0 = z[..., :kout]
    z1 = z[..., kout:2 * kout]
    z2 = z[..., 2 * kout:]
    zero = jnp.zeros((nb, shift, kout), jnp.float32)
    acc = (z1
           + jnp.concatenate([zero, z0[:, :rows - shift]], axis=1)
           + jnp.concatenate([z2[:, shift:], zero], axis=1)
           + b_ref[...])
    if relu:
        acc = jnp.maximum(acc, 0.0)
    return acc


def _fwd_kernel(x_ref, mean_ref, w1_ref, b1_ref, w2_ref, b2_ref,
                w3_ref, b3_ref, o_ref):
    nb = x_ref.shape[0]
    x = x_ref[...]

    lhs = jnp.concatenate([x[:, c] for c in range(C0)], axis=-1)
    lhs = (lhs - mean_ref[...]).astype(jnp.bfloat16)

    a1 = _conv_block(lhs, w1_ref, b1_ref, relu=True).astype(jnp.bfloat16)
    a2 = _conv_block(a1, w2_ref, b2_ref, relu=True).astype(jnp.bfloat16)

    a2_dn = jnp.concatenate([a2[:, 1:], a2[:, :1]], axis=1)
    m = jnp.maximum(a2, a2_dn)
    m_shift = jnp.concatenate([m[..., 1:], m[..., :1]], axis=-1)
    mw = jnp.maximum(m, m_shift)

    a3 = _conv_block(mw, w3_ref, b3_ref, relu=False, shift=2)
    o_ref[...] = a3.astype(o_ref.dtype)


def kernel(x_nchw, mean_l, t1, b1, t2, b2, t3, b3):
    N = x_nchw.shape[0]
    nb = min(_NB, N)
    nblocks = pl.cdiv(N, nb)
    npad = nblocks * nb
    if npad != N:
        x_nchw = jnp.pad(x_nchw, ((0, npad - N), (0, 0), (0, 0), (0, 0)))

    w1 = jnp.concatenate([t1[0], t1[1], t1[2]], axis=1).astype(jnp.bfloat16)
    w2 = jnp.concatenate([t2[0], t2[1], t2[2]], axis=1).astype(jnp.bfloat16)
    w3 = jnp.concatenate([t3[0], t3[1], t3[2]], axis=1).astype(jnp.bfloat16)

    out = pl.pallas_call(
        _fwd_kernel,
        out_shape=jax.ShapeDtypeStruct((npad, H, C2 * WP), x_nchw.dtype),
        grid=(nblocks,),
        in_specs=[
            pl.BlockSpec((nb, C0, H, W), lambda n: (n, 0, 0, 0)),
            pl.BlockSpec((1, 1, C0 * W), lambda n: (0, 0, 0)),
            pl.BlockSpec((C0 * W, 3 * C1 * W), lambda n: (0, 0)),
            pl.BlockSpec((1, C1 * W), lambda n: (0, 0)),
            pl.BlockSpec((C1 * W, 3 * C1 * W), lambda n: (0, 0)),
            pl.BlockSpec((1, C1 * W), lambda n: (0, 0)),
            pl.BlockSpec((C1 * W, 3 * C2 * WP), lambda n: (0, 0)),
            pl.BlockSpec((1, C2 * WP), lambda n: (0, 0)),
        ],
        out_specs=pl.BlockSpec((nb, H, C2 * WP), lambda n: (n, 0, 0)),
        compiler_params=pltpu.CompilerParams(
            dimension_semantics=("parallel",)),
    )(x_nchw, mean_l, w1, b1, w2, b2, w3, b3)

    sel = jnp.eye(H, dtype=x_nchw.dtype)[::2, :]
    return jnp.einsum("nrcw,hr->nchw", out[:N].reshape(N, H, C2, WP), sel)


# speedup vs baseline: 1.4114x; 1.0430x over previous
import jax
import jax.numpy as jnp
from jax.experimental import pallas as pl
from jax.experimental.pallas import tpu as pltpu


H = W = 16
C0, C1, C2 = 3, 8, 16
HP, WP = H // 2, W // 2
_NB = 256


def _extract_taps(t, width):
    cin = t.shape[1] // width
    cout = t.shape[2] // width
    tr = t.reshape(3, cin, width, cout, width)
    return jnp.stack([tr[:, :, dj, :, 1] for dj in range(3)], axis=1)


def _extract_taps_folded(t3f):
    tr = t3f.reshape(3, C1, W, C2, WP)
    return jnp.stack([tr[:, :, 2 * dj, :, 1] for dj in range(3)], axis=1)


def _toeplitz(w, width):
    kh, kw, cin, cout = w.shape
    wi = jnp.arange(width)[:, None]
    wo = jnp.arange(width)[None, :]
    dj = wi - wo + 1
    valid = ((dj >= 0) & (dj < kw)).astype(w.dtype)
    djc = jnp.clip(dj, 0, kw - 1)
    mats = []
    for di in range(kh):
        blk = w[di][djc] * valid[:, :, None, None]
        blk = jnp.transpose(blk, (2, 0, 3, 1))
        mats.append(blk.reshape(cin * width, cout * width))
    return jnp.stack(mats, axis=0)


def _swapped_weights(t1, t2, t3):
    w1 = jnp.transpose(_extract_taps(t1, W), (1, 0, 2, 3))
    w2 = jnp.transpose(_extract_taps(t2, W), (1, 0, 2, 3))
    w3 = jnp.transpose(_extract_taps_folded(t3), (1, 0, 2, 3))
    t1s = _toeplitz(w1, H)
    t2s = _toeplitz(w2, H)
    t3h = _toeplitz(w3, HP)
    rows = jnp.arange(C1 * HP)
    src = (rows // HP) * H + (rows % HP) * 2
    sel = jnp.zeros((C1 * H, C1 * HP), t3h.dtype).at[src, rows].set(1.0)
    t3s = jnp.einsum("sk,dko->dso", sel, t3h)
    cat = lambda t: jnp.concatenate([t[0], t[1], t[2]], axis=1)
    return (cat(t1s).astype(jnp.bfloat16), cat(t2s).astype(jnp.bfloat16),
            cat(t3s).astype(jnp.bfloat16))


def _conv_block(a, w_ref, b_ref, relu, shift=1):
    nb, rows, kin = a.shape
    kout = w_ref.shape[1] // 3
    z = jnp.dot(a.reshape(nb * rows, kin), w_ref[...],
                preferred_element_type=jnp.float32)
    z = z.reshape(nb, rows, 3 * kout)
    z0 = z[..., :kout]
    z1 = z[..., kout:2 * kout]
    z2 = z[..., 2 * kout:]
    zero = jnp.zeros((nb, shift, kout), jnp.float32)
    acc = (z1
           + jnp.concatenate([zero, z0[:, :rows - shift]], axis=1)
           + jnp.concatenate([z2[:, shift:], zero], axis=1)
           + b_ref[...])
    if relu:
        acc = jnp.maximum(acc, 0.0)
    return acc


def _fwd_kernel(x_ref, mean_ref, w1_ref, b1_ref, w2_ref, b2_ref,
                w3_ref, b3_ref, o_ref):
    nb = x_ref.shape[0]
    x = x_ref[...]

    xt = jnp.swapaxes(x.reshape(nb, C0 * H, W), 1, 2)
    lhs = (xt - mean_ref[...]).astype(jnp.bfloat16)

    a1 = _conv_block(lhs, w1_ref, b1_ref, relu=True).astype(jnp.bfloat16)
    a2 = _conv_block(a1, w2_ref, b2_ref, relu=True).astype(jnp.bfloat16)

    a2_dn = jnp.concatenate([a2[:, 1:], a2[:, :1]], axis=1)
    m = jnp.maximum(a2, a2_dn)
    m_shift = jnp.concatenate([m[..., 1:], m[..., :1]], axis=-1)
    mw = jnp.maximum(m, m_shift)

    a3 = _conv_block(mw, w3_ref, b3_ref, relu=False, shift=2)
    o_ref[...] = a3.astype(o_ref.dtype)


def kernel(x_nchw, mean_l, t1, b1, t2, b2, t3, b3):
    N = x_nchw.shape[0]
    nb = min(_NB, N)
    nblocks = pl.cdiv(N, nb)
    npad = nblocks * nb
    if npad != N:
        x_nchw = jnp.pad(x_nchw, ((0, npad - N), (0, 0), (0, 0), (0, 0)))

    w1, w2, w3 = _swapped_weights(t1, t2, t3)

    out = pl.pallas_call(
        _fwd_kernel,
        out_shape=jax.ShapeDtypeStruct((npad, W, C2 * HP), x_nchw.dtype),
        grid=(nblocks,),
        in_specs=[
            pl.BlockSpec((nb, C0, H, W), lambda n: (n, 0, 0, 0)),
            pl.BlockSpec((1, 1, C0 * H), lambda n: (0, 0, 0)),
            pl.BlockSpec((C0 * H, 3 * C1 * H), lambda n: (0, 0)),
            pl.BlockSpec((1, C1 * H), lambda n: (0, 0)),
            pl.BlockSpec((C1 * H, 3 * C1 * H), lambda n: (0, 0)),
            pl.BlockSpec((1, C1 * H), lambda n: (0, 0)),
            pl.BlockSpec((C1 * H, 3 * C2 * HP), lambda n: (0, 0)),
            pl.BlockSpec((1, C2 * HP), lambda n: (0, 0)),
        ],
        out_specs=pl.BlockSpec((nb, W, C2 * HP), lambda n: (n, 0, 0)),
        compiler_params=pltpu.CompilerParams(
            dimension_semantics=("parallel",)),
    )(x_nchw, mean_l, w1, b1, w2, b2, w3, b3)

    sel = jnp.eye(W, dtype=x_nchw.dtype)[::2, :]
    return jnp.einsum("nvch,wv->nchw", out[:N].reshape(N, W, C2, HP), sel)


# speedup vs baseline: 1.9086x; 1.3523x over previous
import jax
import jax.numpy as jnp
from jax.experimental import pallas as pl
from jax.experimental.pallas import tpu as pltpu


H = W = 16
C0, C1, C2 = 3, 8, 16
HP, WP = H // 2, W // 2
_NB = 256


def _extract_taps(t, width):
    cin = t.shape[1] // width
    cout = t.shape[2] // width
    tr = t.reshape(3, cin, width, cout, width)
    return jnp.stack([tr[:, :, dj, :, 1] for dj in range(3)], axis=1)


def _extract_taps_folded(t3f):
    tr = t3f.reshape(3, C1, W, C2, WP)
    return jnp.stack([tr[:, :, 2 * dj, :, 1] for dj in range(3)], axis=1)


def _toeplitz(w, width):
    kh, kw, cin, cout = w.shape
    wi = jnp.arange(width)[:, None]
    wo = jnp.arange(width)[None, :]
    dj = wi - wo + 1
    valid = ((dj >= 0) & (dj < kw)).astype(w.dtype)
    djc = jnp.clip(dj, 0, kw - 1)
    mats = []
    for di in range(kh):
        blk = w[di][djc] * valid[:, :, None, None]
        blk = jnp.transpose(blk, (2, 0, 3, 1))
        mats.append(blk.reshape(cin * width, cout * width))
    return jnp.stack(mats, axis=0)


def _swapped_weights(t1, t2, t3):
    w1 = jnp.transpose(_extract_taps(t1, W), (1, 0, 2, 3))
    w2 = jnp.transpose(_extract_taps(t2, W), (1, 0, 2, 3))
    w3 = jnp.transpose(_extract_taps_folded(t3), (1, 0, 2, 3))
    t1s = _toeplitz(w1, H)
    t2s = _toeplitz(w2, H)
    t3h = _toeplitz(w3, HP)
    rows = jnp.arange(C1 * HP)
    src = (rows // HP) * H + (rows % HP) * 2
    sel = jnp.zeros((C1 * H, C1 * HP), t3h.dtype).at[src, rows].set(1.0)
    t3s = jnp.einsum("sk,dko->dso", sel, t3h)
    cat = lambda t: jnp.concatenate([t[0], t[1], t[2]], axis=1)
    return (cat(t1s).astype(jnp.bfloat16), cat(t2s).astype(jnp.bfloat16),
            cat(t3s).astype(jnp.bfloat16))


def _conv_block(a, w_ref, b_ref, relu, shift=1, f32_tail=False):
    nb, rows, kin = a.shape
    kout = w_ref.shape[1] // 3
    z = jnp.dot(a.reshape(nb * rows, kin), w_ref[...],
                preferred_element_type=jnp.float32)
    if not f32_tail:
        z = z.astype(jnp.bfloat16)
    z = z.reshape(nb, rows, 3 * kout)
    z0 = z[..., :kout]
    z1 = z[..., kout:2 * kout]
    z2 = z[..., 2 * kout:]
    zero = jnp.zeros((nb, shift, kout), z.dtype)
    acc = (z1
           + jnp.concatenate([zero, z0[:, :rows - shift]], axis=1)
           + jnp.concatenate([z2[:, shift:], zero], axis=1)
           + b_ref[...].astype(z.dtype))
    if relu:
        acc = jnp.maximum(acc, 0.0)
    return acc


def _fwd_kernel(x_ref, mean_ref, w1_ref, b1_ref, w2_ref, b2_ref,
                w3_ref, b3_ref, o_ref):
    nb = x_ref.shape[0]
    lhs = x_ref[...] - mean_ref[...]

    a1 = _conv_block(lhs, w1_ref, b1_ref, relu=True)
    a2 = _conv_block(a1, w2_ref, b2_ref, relu=True)

    a2_dn = jnp.concatenate([a2[:, 1:], a2[:, :1]], axis=1)
    m = jnp.maximum(a2, a2_dn)
    m_shift = jnp.concatenate([m[..., 1:], m[..., :1]], axis=-1)
    mw = jnp.maximum(m, m_shift)

    a3 = _conv_block(mw, w3_ref, b3_ref, relu=False, shift=2,
                     f32_tail=True)
    o_ref[...] = a3.astype(o_ref.dtype)


def kernel(x_nchw, mean_l, t1, b1, t2, b2, t3, b3):
    N = x_nchw.shape[0]
    nb = min(_NB, N)
    nblocks = pl.cdiv(N, nb)
    npad = nblocks * nb
    if npad != N:
        x_nchw = jnp.pad(x_nchw, ((0, npad - N), (0, 0), (0, 0), (0, 0)))

    w1, w2, w3 = _swapped_weights(t1, t2, t3)

    xt = jnp.transpose(x_nchw, (0, 3, 1, 2)).reshape(npad, W, C0 * H)
    xt = xt.astype(jnp.bfloat16)
    mean_b = mean_l.astype(jnp.bfloat16)

    out = pl.pallas_call(
        _fwd_kernel,
        out_shape=jax.ShapeDtypeStruct((npad, W, C2 * HP), x_nchw.dtype),
        grid=(nblocks,),
        in_specs=[
            pl.BlockSpec((nb, W, C0 * H), lambda n: (n, 0, 0)),
            pl.BlockSpec((1, 1, C0 * H), lambda n: (0, 0, 0)),
            pl.BlockSpec((C0 * H, 3 * C1 * H), lambda n: (0, 0)),
            pl.BlockSpec((1, C1 * H), lambda n: (0, 0)),
            pl.BlockSpec((C1 * H, 3 * C1 * H), lambda n: (0, 0)),
            pl.BlockSpec((1, C1 * H), lambda n: (0, 0)),
            pl.BlockSpec((C1 * H, 3 * C2 * HP), lambda n: (0, 0)),
            pl.BlockSpec((1, C2 * HP), lambda n: (0, 0)),
        ],
        out_specs=pl.BlockSpec((nb, W, C2 * HP), lambda n: (n, 0, 0)),
        compiler_params=pltpu.CompilerParams(
            dimension_semantics=("parallel",)),
    )(xt, mean_b, w1, b1, w2, b2, w3, b3)

    sel = jnp.eye(W, dtype=x_nchw.dtype)[::2, :]
    return jnp.einsum("nvch,wv->nchw", out[:N].reshape(N, W, C2, HP), sel)


# speedup vs baseline: 1.9642x; 1.0291x over previous
import jax
import jax.numpy as jnp
from jax.experimental import pallas as pl
from jax.experimental.pallas import tpu as pltpu


H = W = 16
C0, C1, C2 = 3, 8, 16
HP, WP = H // 2, W // 2
_NB = 256


def _extract_taps(t, width):
    cin = t.shape[1] // width
    cout = t.shape[2] // width
    tr = t.reshape(3, cin, width, cout, width)
    return jnp.stack([tr[:, :, dj, :, 1] for dj in range(3)], axis=1)


def _extract_taps_folded(t3f):
    tr = t3f.reshape(3, C1, W, C2, WP)
    return jnp.stack([tr[:, :, 2 * dj, :, 1] for dj in range(3)], axis=1)


def _toeplitz(w, width):
    kh, kw, cin, cout = w.shape
    wi = jnp.arange(width)[:, None]
    wo = jnp.arange(width)[None, :]
    dj = wi - wo + 1
    valid = ((dj >= 0) & (dj < kw)).astype(w.dtype)
    djc = jnp.clip(dj, 0, kw - 1)
    mats = []
    for di in range(kh):
        blk = w[di][djc] * valid[:, :, None, None]
        blk = jnp.transpose(blk, (2, 0, 3, 1))
        mats.append(blk.reshape(cin * width, cout * width))
    return jnp.stack(mats, axis=0)


def _swapped_weights(t1, t2, t3):
    w1 = jnp.transpose(_extract_taps(t1, W), (1, 0, 2, 3))
    w2 = jnp.transpose(_extract_taps(t2, W), (1, 0, 2, 3))
    w3 = jnp.transpose(_extract_taps_folded(t3), (1, 0, 2, 3))
    t1s = _toeplitz(w1, H)
    t2s = _toeplitz(w2, H)
    t3h = _toeplitz(w3, HP)
    rows = jnp.arange(C1 * HP)
    src = (rows // HP) * H + (rows % HP) * 2
    sel = jnp.zeros((C1 * H, C1 * HP), t3h.dtype).at[src, rows].set(1.0)
    t3s = jnp.einsum("sk,dko->dso", sel, t3h)
    cat = lambda t: jnp.concatenate([t[0], t[1], t[2]], axis=1)
    return (cat(t1s).astype(jnp.bfloat16), cat(t2s).astype(jnp.bfloat16),
            cat(t3s).astype(jnp.bfloat16))


def _conv_block(a, w_ref, b_ref, relu, shift=1, f32_tail=False):
    nb, rows, kin = a.shape
    kout = w_ref.shape[1] // 3
    z = jnp.dot(a.reshape(nb * rows, kin), w_ref[...],
                preferred_element_type=jnp.float32)
    if not f32_tail:
        z = z.astype(jnp.bfloat16)
    z = z.reshape(nb, rows, 3 * kout)
    z0 = z[..., :kout]
    z1 = z[..., kout:2 * kout]
    z2 = z[..., 2 * kout:]
    zero = jnp.zeros((nb, shift, kout), z.dtype)
    acc = (z1
           + jnp.concatenate([zero, z0[:, :rows - shift]], axis=1)
           + jnp.concatenate([z2[:, shift:], zero], axis=1)
           + b_ref[...].astype(z.dtype))
    if relu:
        acc = jnp.maximum(acc, 0.0)
    return acc


def _fwd_kernel(x_ref, mean_ref, w1_ref, b1_ref, w2_ref, b2_ref,
                w3_ref, b3_ref, o_ref):
    nb = x_ref.shape[0]
    lhs = x_ref[...] - mean_ref[...]

    a1 = _conv_block(lhs, w1_ref, b1_ref, relu=True,
                     f32_tail=True).astype(jnp.bfloat16)
    a2 = _conv_block(a1, w2_ref, b2_ref, relu=True,
                     f32_tail=True).astype(jnp.bfloat16)

    a2_dn = jnp.concatenate([a2[:, 1:], a2[:, :1]], axis=1)
    m = jnp.maximum(a2, a2_dn)
    m_shift = jnp.concatenate([m[..., 1:], m[..., :1]], axis=-1)
    mw = jnp.maximum(m, m_shift)

    a3 = _conv_block(mw, w3_ref, b3_ref, relu=False, shift=2)
    o_ref[...] = a3.astype(o_ref.dtype)


def kernel(x_nchw, mean_l, t1, b1, t2, b2, t3, b3):
    N = x_nchw.shape[0]
    nb = min(_NB, N)
    nblocks = pl.cdiv(N, nb)
    npad = nblocks * nb
    if npad != N:
        x_nchw = jnp.pad(x_nchw, ((0, npad - N), (0, 0), (0, 0), (0, 0)))

    w1, w2, w3 = _swapped_weights(t1, t2, t3)

    xt = jax.lax.reshape(x_nchw, (npad, W, C0 * H), dimensions=(0, 3, 1, 2))
    xt = xt.astype(jnp.bfloat16)
    mean_b = mean_l.astype(jnp.bfloat16)

    out = pl.pallas_call(
        _fwd_kernel,
        out_shape=jax.ShapeDtypeStruct((npad, W, C2 * HP), jnp.bfloat16),
        grid=(nblocks,),
        in_specs=[
            pl.BlockSpec((nb, W, C0 * H), lambda n: (n, 0, 0)),
            pl.BlockSpec((1, 1, C0 * H), lambda n: (0, 0, 0)),
            pl.BlockSpec((C0 * H, 3 * C1 * H), lambda n: (0, 0)),
            pl.BlockSpec((1, C1 * H), lambda n: (0, 0)),
            pl.BlockSpec((C1 * H, 3 * C1 * H), lambda n: (0, 0)),
            pl.BlockSpec((1, C1 * H), lambda n: (0, 0)),
            pl.BlockSpec((C1 * H, 3 * C2 * HP), lambda n: (0, 0)),
            pl.BlockSpec((1, C2 * HP), lambda n: (0, 0)),
        ],
        out_specs=pl.BlockSpec((nb, W, C2 * HP), lambda n: (n, 0, 0)),
        compiler_params=pltpu.CompilerParams(
            dimension_semantics=("parallel",)),
    )(xt, mean_b, w1, b1, w2, b2, w3, b3)

    sel = jnp.eye(W, dtype=jnp.bfloat16)[::2, :]
    return jnp.einsum("nvch,wv->nchw", out[:N].reshape(N, W, C2, HP), sel,
                      preferred_element_type=x_nchw.dtype)


# speedup vs baseline: 1.9656x; 1.0007x over previous
import jax
import jax.numpy as jnp
from jax.experimental import pallas as pl
from jax.experimental.pallas import tpu as pltpu


H = W = 16
C0, C1, C2 = 3, 8, 16
HP, WP = H // 2, W // 2
_NB = 256


def _extract_taps(t, width):
    cin = t.shape[1] // width
    cout = t.shape[2] // width
    tr = t.reshape(3, cin, width, cout, width)
    return jnp.stack([tr[:, :, dj, :, 1] for dj in range(3)], axis=1)


def _extract_taps_folded(t3f):
    tr = t3f.reshape(3, C1, W, C2, WP)
    return jnp.stack([tr[:, :, 2 * dj, :, 1] for dj in range(3)], axis=1)


def _toeplitz(w, width):
    kh, kw, cin, cout = w.shape
    wi = jnp.arange(width)[:, None]
    wo = jnp.arange(width)[None, :]
    dj = wi - wo + 1
    valid = ((dj >= 0) & (dj < kw)).astype(w.dtype)
    djc = jnp.clip(dj, 0, kw - 1)
    mats = []
    for di in range(kh):
        blk = w[di][djc] * valid[:, :, None, None]
        blk = jnp.transpose(blk, (2, 0, 3, 1))
        mats.append(blk.reshape(cin * width, cout * width))
    return jnp.stack(mats, axis=0)


def _swapped_weights(t1, t2, t3):
    w1 = jnp.transpose(_extract_taps(t1, W), (1, 0, 2, 3))
    w2 = jnp.transpose(_extract_taps(t2, W), (1, 0, 2, 3))
    w3 = jnp.transpose(_extract_taps_folded(t3), (1, 0, 2, 3))
    t1s = _toeplitz(w1, H)
    t2s = _toeplitz(w2, H)
    t3h = _toeplitz(w3, HP)
    rows = jnp.arange(C1 * HP)
    src = (rows // HP) * H + (rows % HP) * 2
    sel = jnp.zeros((C1 * H, C1 * HP), t3h.dtype).at[src, rows].set(1.0)
    t3s = jnp.einsum("sk,dko->dso", sel, t3h)
    cat = lambda t: jnp.concatenate([t[0], t[1], t[2]], axis=1)
    return (cat(t1s).astype(jnp.bfloat16), cat(t2s).astype(jnp.bfloat16),
            cat(t3s).astype(jnp.bfloat16))


def _conv_block(a, w_ref, b_ref, relu, shift=1, f32_tail=False):
    nb, rows, kin = a.shape
    kout = w_ref.shape[1] // 3
    z = jnp.dot(a.reshape(nb * rows, kin), w_ref[...],
                preferred_element_type=jnp.float32)
    if not f32_tail:
        z = z.astype(jnp.bfloat16)
    z = z.reshape(nb, rows, 3 * kout)
    z0 = z[..., :kout]
    z1 = z[..., kout:2 * kout]
    z2 = z[..., 2 * kout:]
    zero = jnp.zeros((nb, shift, kout), z.dtype)
    acc = (z1
           + jnp.concatenate([zero, z0[:, :rows - shift]], axis=1)
           + jnp.concatenate([z2[:, shift:], zero], axis=1)
           + b_ref[...].astype(z.dtype))
    if relu:
        acc = jnp.maximum(acc, 0.0)
    return acc


def _fwd_kernel(x_ref, mean_ref, w1_ref, b1_ref, w2_ref, b2_ref,
                w3_ref, b3_ref, o_ref):
    nb = x_ref.shape[0]
    lhs = x_ref[...] - mean_ref[...]

    a1 = _conv_block(lhs, w1_ref, b1_ref, relu=True,
                     f32_tail=True).astype(jnp.bfloat16)
    a2 = _conv_block(a1, w2_ref, b2_ref, relu=True,
                     f32_tail=True).astype(jnp.bfloat16)

    a2_dn = jnp.concatenate([a2[:, 1:], a2[:, :1]], axis=1)
    m = jnp.maximum(a2, a2_dn)
    m_shift = jnp.concatenate([m[..., 1:], m[..., :1]], axis=-1)
    mw = jnp.maximum(m, m_shift)

    a3 = _conv_block(mw, w3_ref, b3_ref, relu=False, shift=2)
    o_ref[...] = a3.astype(o_ref.dtype)


def kernel(x_nchw, mean_l, t1, b1, t2, b2, t3, b3):
    N = x_nchw.shape[0]
    nb = min(_NB, N)
    nblocks = pl.cdiv(N, nb)
    npad = nblocks * nb
    if npad != N:
        x_nchw = jnp.pad(x_nchw, ((0, npad - N), (0, 0), (0, 0), (0, 0)))

    w1, w2, w3 = _swapped_weights(t1, t2, t3)

    xt = jax.lax.reshape(x_nchw.astype(jnp.bfloat16), (npad, W, C0 * H),
                         dimensions=(0, 3, 1, 2))
    mean_b = mean_l.astype(jnp.bfloat16)

    out = pl.pallas_call(
        _fwd_kernel,
        out_shape=jax.ShapeDtypeStruct((npad, W, C2 * HP), jnp.bfloat16),
        grid=(nblocks,),
        in_specs=[
            pl.BlockSpec((nb, W, C0 * H), lambda n: (n, 0, 0)),
            pl.BlockSpec((1, 1, C0 * H), lambda n: (0, 0, 0)),
            pl.BlockSpec((C0 * H, 3 * C1 * H), lambda n: (0, 0)),
            pl.BlockSpec((1, C1 * H), lambda n: (0, 0)),
            pl.BlockSpec((C1 * H, 3 * C1 * H), lambda n: (0, 0)),
            pl.BlockSpec((1, C1 * H), lambda n: (0, 0)),
            pl.BlockSpec((C1 * H, 3 * C2 * HP), lambda n: (0, 0)),
            pl.BlockSpec((1, C2 * HP), lambda n: (0, 0)),
        ],
        out_specs=pl.BlockSpec((nb, W, C2 * HP), lambda n: (n, 0, 0)),
        compiler_params=pltpu.CompilerParams(
            dimension_semantics=("parallel",)),
    )(xt, mean_b, w1, b1, w2, b2, w3, b3)

    sel = jnp.eye(W, dtype=jnp.bfloat16)[::2, :]
    return jnp.einsum("nvch,wv->nchw", out[:N].reshape(N, W, C2, HP), sel,
                      preferred_element_type=x_nchw.dtype)


# speedup vs baseline: 1.9854x; 1.0101x over previous
import jax
import jax.numpy as jnp
from jax.experimental import pallas as pl
from jax.experimental.pallas import tpu as pltpu


H = W = 16
C0, C1, C2 = 3, 8, 16
HP, WP = H // 2, W // 2
_NB = 512


def _extract_taps(t, width):
    cin = t.shape[1] // width
    cout = t.shape[2] // width
    tr = t.reshape(3, cin, width, cout, width)
    return jnp.stack([tr[:, :, dj, :, 1] for dj in range(3)], axis=1)


def _extract_taps_folded(t3f):
    tr = t3f.reshape(3, C1, W, C2, WP)
    return jnp.stack([tr[:, :, 2 * dj, :, 1] for dj in range(3)], axis=1)


def _toeplitz(w, width):
    kh, kw, cin, cout = w.shape
    wi = jnp.arange(width)[:, None]
    wo = jnp.arange(width)[None, :]
    dj = wi - wo + 1
    valid = ((dj >= 0) & (dj < kw)).astype(w.dtype)
    djc = jnp.clip(dj, 0, kw - 1)
    mats = []
    for di in range(kh):
        blk = w[di][djc] * valid[:, :, None, None]
        blk = jnp.transpose(blk, (2, 0, 3, 1))
        mats.append(blk.reshape(cin * width, cout * width))
    return jnp.stack(mats, axis=0)


def _swapped_weights(t1, t2, t3):
    w1 = jnp.transpose(_extract_taps(t1, W), (1, 0, 2, 3))
    w2 = jnp.transpose(_extract_taps(t2, W), (1, 0, 2, 3))
    w3 = jnp.transpose(_extract_taps_folded(t3), (1, 0, 2, 3))
    t1s = _toeplitz(w1, H)
    t2s = _toeplitz(w2, H)
    t3h = _toeplitz(w3, HP)
    rows = jnp.arange(C1 * HP)
    src = (rows // HP) * H + (rows % HP) * 2
    sel = jnp.zeros((C1 * H, C1 * HP), t3h.dtype).at[src, rows].set(1.0)
    t3s = jnp.einsum("sk,dko->dso", sel, t3h)
    cat = lambda t: jnp.concatenate([t[0], t[1], t[2]], axis=1)
    return (cat(t1s).astype(jnp.bfloat16), cat(t2s).astype(jnp.bfloat16),
            cat(t3s).astype(jnp.bfloat16))


def _conv_block(a, w_ref, b_ref, relu, shift=1, f32_tail=False):
    nb, rows, kin = a.shape
    kout = w_ref.shape[1] // 3
    z = jnp.dot(a.reshape(nb * rows, kin), w_ref[...],
                preferred_element_type=jnp.float32)
    if not f32_tail:
        z = z.astype(jnp.bfloat16)
    z = z.reshape(nb, rows, 3 * kout)
    z0 = z[..., :kout]
    z1 = z[..., kout:2 * kout]
    z2 = z[..., 2 * kout:]
    zero = jnp.zeros((nb, shift, kout), z.dtype)
    acc = (z1
           + jnp.concatenate([zero, z0[:, :rows - shift]], axis=1)
           + jnp.concatenate([z2[:, shift:], zero], axis=1)
           + b_ref[...].astype(z.dtype))
    if relu:
        acc = jnp.maximum(acc, 0.0)
    return acc


def _fwd_kernel(x_ref, mean_ref, w1_ref, b1_ref, w2_ref, b2_ref,
                w3_ref, b3_ref, o_ref):
    nb = x_ref.shape[0]
    lhs = x_ref[...] - mean_ref[...]

    a1 = _conv_block(lhs, w1_ref, b1_ref, relu=True,
                     f32_tail=True).astype(jnp.bfloat16)
    a2 = _conv_block(a1, w2_ref, b2_ref, relu=True,
                     f32_tail=True).astype(jnp.bfloat16)

    a2_dn = jnp.concatenate([a2[:, 1:], a2[:, :1]], axis=1)
    m = jnp.maximum(a2, a2_dn)
    m_shift = jnp.concatenate([m[..., 1:], m[..., :1]], axis=-1)
    mw = jnp.maximum(m, m_shift)

    a3 = _conv_block(mw, w3_ref, b3_ref, relu=False, shift=2)
    o_ref[...] = a3.astype(o_ref.dtype)


def kernel(x_nchw, mean_l, t1, b1, t2, b2, t3, b3):
    N = x_nchw.shape[0]
    nb = min(_NB, N)
    nblocks = pl.cdiv(N, nb)
    npad = nblocks * nb
    if npad != N:
        x_nchw = jnp.pad(x_nchw, ((0, npad - N), (0, 0), (0, 0), (0, 0)))

    w1, w2, w3 = _swapped_weights(t1, t2, t3)

    xt = jax.lax.reshape(x_nchw.astype(jnp.bfloat16), (npad, W, C0 * H),
                         dimensions=(0, 3, 1, 2))
    mean_b = mean_l.astype(jnp.bfloat16)

    out = pl.pallas_call(
        _fwd_kernel,
        out_shape=jax.ShapeDtypeStruct((npad, W, C2 * HP), jnp.bfloat16),
        grid=(nblocks,),
        in_specs=[
            pl.BlockSpec((nb, W, C0 * H), lambda n: (n, 0, 0)),
            pl.BlockSpec((1, 1, C0 * H), lambda n: (0, 0, 0)),
            pl.BlockSpec((C0 * H, 3 * C1 * H), lambda n: (0, 0)),
            pl.BlockSpec((1, C1 * H), lambda n: (0, 0)),
            pl.BlockSpec((C1 * H, 3 * C1 * H), lambda n: (0, 0)),
            pl.BlockSpec((1, C1 * H), lambda n: (0, 0)),
            pl.BlockSpec((C1 * H, 3 * C2 * HP), lambda n: (0, 0)),
            pl.BlockSpec((1, C2 * HP), lambda n: (0, 0)),
        ],
        out_specs=pl.BlockSpec((nb, W, C2 * HP), lambda n: (n, 0, 0)),
        compiler_params=pltpu.CompilerParams(
            dimension_semantics=("parallel",)),
    )(xt, mean_b, w1, b1, w2, b2, w3, b3)

    sel = jnp.eye(W, dtype=jnp.bfloat16)[::2, :]
    return jnp.einsum("nvch,wv->nchw", out[:N].reshape(N, W, C2, HP), sel,
                      preferred_element_type=x_nchw.dtype)


# speedup vs baseline: 1.9986x; 1.0066x over previous
import jax
import jax.numpy as jnp
from jax.experimental import pallas as pl
from jax.experimental.pallas import tpu as pltpu


H = W = 16
C0, C1, C2 = 3, 8, 16
HP, WP = H // 2, W // 2
_NB = 1024


def _extract_taps(t, width):
    cin = t.shape[1] // width
    cout = t.shape[2] // width
    tr = t.reshape(3, cin, width, cout, width)
    return jnp.stack([tr[:, :, dj, :, 1] for dj in range(3)], axis=1)


def _extract_taps_folded(t3f):
    tr = t3f.reshape(3, C1, W, C2, WP)
    return jnp.stack([tr[:, :, 2 * dj, :, 1] for dj in range(3)], axis=1)


def _toeplitz(w, width):
    kh, kw, cin, cout = w.shape
    wi = jnp.arange(width)[:, None]
    wo = jnp.arange(width)[None, :]
    dj = wi - wo + 1
    valid = ((dj >= 0) & (dj < kw)).astype(w.dtype)
    djc = jnp.clip(dj, 0, kw - 1)
    mats = []
    for di in range(kh):
        blk = w[di][djc] * valid[:, :, None, None]
        blk = jnp.transpose(blk, (2, 0, 3, 1))
        mats.append(blk.reshape(cin * width, cout * width))
    return jnp.stack(mats, axis=0)


def _swapped_weights(t1, t2, t3):
    w1 = jnp.transpose(_extract_taps(t1, W), (1, 0, 2, 3))
    w2 = jnp.transpose(_extract_taps(t2, W), (1, 0, 2, 3))
    w3 = jnp.transpose(_extract_taps_folded(t3), (1, 0, 2, 3))
    t1s = _toeplitz(w1, H)
    t2s = _toeplitz(w2, H)
    t3h = _toeplitz(w3, HP)
    rows = jnp.arange(C1 * HP)
    src = (rows // HP) * H + (rows % HP) * 2
    sel = jnp.zeros((C1 * H, C1 * HP), t3h.dtype).at[src, rows].set(1.0)
    t3s = jnp.einsum("sk,dko->dso", sel, t3h)
    cat = lambda t: jnp.concatenate([t[0], t[1], t[2]], axis=1)
    return (cat(t1s).astype(jnp.bfloat16), cat(t2s).astype(jnp.bfloat16),
            cat(t3s).astype(jnp.bfloat16))


def _conv_block(a, w_ref, b_ref, relu, shift=1, f32_tail=False):
    nb, rows, kin = a.shape
    kout = w_ref.shape[1] // 3
    z = jnp.dot(a.reshape(nb * rows, kin), w_ref[...],
                preferred_element_type=jnp.float32)
    if not f32_tail:
        z = z.astype(jnp.bfloat16)
    z = z.reshape(nb, rows, 3 * kout)
    z0 = z[..., :kout]
    z1 = z[..., kout:2 * kout]
    z2 = z[..., 2 * kout:]
    zero = jnp.zeros((nb, shift, kout), z.dtype)
    acc = (z1
           + jnp.concatenate([zero, z0[:, :rows - shift]], axis=1)
           + jnp.concatenate([z2[:, shift:], zero], axis=1)
           + b_ref[...].astype(z.dtype))
    if relu:
        acc = jnp.maximum(acc, 0.0)
    return acc


def _fwd_kernel(x_ref, mean_ref, w1_ref, b1_ref, w2_ref, b2_ref,
                w3_ref, b3_ref, o_ref):
    nb = x_ref.shape[0]
    lhs = x_ref[...] - mean_ref[...]

    a1 = _conv_block(lhs, w1_ref, b1_ref, relu=True,
                     f32_tail=True).astype(jnp.bfloat16)
    a2 = _conv_block(a1, w2_ref, b2_ref, relu=True,
                     f32_tail=True).astype(jnp.bfloat16)

    a2_dn = jnp.concatenate([a2[:, 1:], a2[:, :1]], axis=1)
    m = jnp.maximum(a2, a2_dn)
    m_shift = jnp.concatenate([m[..., 1:], m[..., :1]], axis=-1)
    mw = jnp.maximum(m, m_shift)

    a3 = _conv_block(mw, w3_ref, b3_ref, relu=False, shift=2)
    o_ref[...] = a3.astype(o_ref.dtype)


def kernel(x_nchw, mean_l, t1, b1, t2, b2, t3, b3):
    N = x_nchw.shape[0]
    nb = min(_NB, N)
    nblocks = pl.cdiv(N, nb)
    npad = nblocks * nb
    if npad != N:
        x_nchw = jnp.pad(x_nchw, ((0, npad - N), (0, 0), (0, 0), (0, 0)))

    w1, w2, w3 = _swapped_weights(t1, t2, t3)

    xt = jax.lax.reshape(x_nchw.astype(jnp.bfloat16), (npad, W, C0 * H),
                         dimensions=(0, 3, 1, 2))
    mean_b = mean_l.astype(jnp.bfloat16)

    out = pl.pallas_call(
        _fwd_kernel,
        out_shape=jax.ShapeDtypeStruct((npad, W, C2 * HP), jnp.bfloat16),
        grid=(nblocks,),
        in_specs=[
            pl.BlockSpec((nb, W, C0 * H), lambda n: (n, 0, 0)),
            pl.BlockSpec((1, 1, C0 * H), lambda n: (0, 0, 0)),
            pl.BlockSpec((C0 * H, 3 * C1 * H), lambda n: (0, 0)),
            pl.BlockSpec((1, C1 * H), lambda n: (0, 0)),
            pl.BlockSpec((C1 * H, 3 * C1 * H), lambda n: (0, 0)),
            pl.BlockSpec((1, C1 * H), lambda n: (0, 0)),
            pl.BlockSpec((C1 * H, 3 * C2 * HP), lambda n: (0, 0)),
            pl.BlockSpec((1, C2 * HP), lambda n: (0, 0)),
        ],
        out_specs=pl.BlockSpec((nb, W, C2 * HP), lambda n: (n, 0, 0)),
        compiler_params=pltpu.CompilerParams(
            dimension_semantics=("parallel",)),
    )(xt, mean_b, w1, b1, w2, b2, w3, b3)

    sel = jnp.eye(W, dtype=jnp.bfloat16)[::2, :]
    return jnp.einsum("nvch,wv->nchw", out[:N].reshape(N, W, C2, HP), sel,
                      preferred_element_type=x_nchw.dtype)


# speedup vs baseline: 1.9991x; 1.0003x over previous
import jax
import jax.numpy as jnp
from jax.experimental import pallas as pl
from jax.experimental.pallas import tpu as pltpu


H = W = 16
C0, C1, C2 = 3, 8, 16
HP, WP = H // 2, W // 2
_NB = 1024


def _extract_taps(t, width):
    cin = t.shape[1] // width
    cout = t.shape[2] // width
    tr = t.reshape(3, cin, width, cout, width)
    return jnp.stack([tr[:, :, dj, :, 1] for dj in range(3)], axis=1)


def _extract_taps_folded(t3f):
    tr = t3f.reshape(3, C1, W, C2, WP)
    return jnp.stack([tr[:, :, 2 * dj, :, 1] for dj in range(3)], axis=1)


def _toeplitz(w, width):
    kh, kw, cin, cout = w.shape
    wi = jnp.arange(width)[:, None]
    wo = jnp.arange(width)[None, :]
    dj = wi - wo + 1
    valid = ((dj >= 0) & (dj < kw)).astype(w.dtype)
    djc = jnp.clip(dj, 0, kw - 1)
    mats = []
    for di in range(kh):
        blk = w[di][djc] * valid[:, :, None, None]
        blk = jnp.transpose(blk, (2, 0, 3, 1))
        mats.append(blk.reshape(cin * width, cout * width))
    return jnp.stack(mats, axis=0)


def _swapped_weights(t1, t2, t3):
    w1 = jnp.transpose(_extract_taps(t1, W), (1, 0, 2, 3))
    w2 = jnp.transpose(_extract_taps(t2, W), (1, 0, 2, 3))
    w3 = jnp.transpose(_extract_taps_folded(t3), (1, 0, 2, 3))
    t1s = _toeplitz(w1, H)
    t2s = _toeplitz(w2, H)
    t3h = _toeplitz(w3, HP)
    rows = jnp.arange(C1 * HP)
    src = (rows // HP) * H + (rows % HP) * 2
    sel = jnp.zeros((C1 * H, C1 * HP), t3h.dtype).at[src, rows].set(1.0)
    t3s = jnp.einsum("sk,dko->dso", sel, t3h)
    cat = lambda t: jnp.concatenate([t[0], t[1], t[2]], axis=1)
    return (cat(t1s).astype(jnp.bfloat16), cat(t2s).astype(jnp.bfloat16),
            cat(t3s).astype(jnp.bfloat16))


def _conv_block(a, w_ref, b_ref, relu, shift=1, f32_tail=False):
    nb, rows, kin = a.shape
    kout = w_ref.shape[1] // 3
    z = jnp.dot(a.reshape(nb * rows, kin), w_ref[...],
                preferred_element_type=jnp.float32)
    if not f32_tail:
        z = z.astype(jnp.bfloat16)
    z = z.reshape(nb, rows, 3 * kout)
    z0 = z[..., :kout]
    z1 = z[..., kout:2 * kout]
    z2 = z[..., 2 * kout:]
    zero = jnp.zeros((nb, shift, kout), z.dtype)
    acc = (z1
           + jnp.concatenate([zero, z0[:, :rows - shift]], axis=1)
           + jnp.concatenate([z2[:, shift:], zero], axis=1)
           + b_ref[...].astype(z.dtype))
    if relu:
        acc = jnp.maximum(acc, 0.0)
    return acc


def _fwd_kernel(x_ref, mean_ref, w1_ref, b1_ref, w2_ref, b2_ref,
                w3_ref, b3_ref, o_ref):
    nb = x_ref.shape[0]
    lhs = x_ref[...] - mean_ref[...]

    a1 = _conv_block(lhs, w1_ref, b1_ref, relu=True,
                     f32_tail=True).astype(jnp.bfloat16)
    a2 = _conv_block(a1, w2_ref, b2_ref, relu=True,
                     f32_tail=True).astype(jnp.bfloat16)

    a2_dn = jnp.concatenate([a2[:, 1:], a2[:, :1]], axis=1)
    m = jnp.maximum(a2, a2_dn)
    m_shift = jnp.concatenate([m[..., 1:], m[..., :1]], axis=-1)
    mw = jnp.maximum(m, m_shift)

    a3 = _conv_block(mw, w3_ref, b3_ref, relu=False, shift=2)
    o_ref[...] = a3.astype(o_ref.dtype)


def kernel(x_nchw, mean_l, t1, b1, t2, b2, t3, b3):
    N = x_nchw.shape[0]
    nb = min(_NB, N)
    nblocks = pl.cdiv(N, nb)
    npad = nblocks * nb
    if npad != N:
        x_nchw = jnp.pad(x_nchw, ((0, npad - N), (0, 0), (0, 0), (0, 0)))

    w1, w2, w3 = _swapped_weights(t1, t2, t3)

    xt = jax.lax.reshape(x_nchw.astype(jnp.bfloat16), (npad, W, C0 * H),
                         dimensions=(0, 3, 1, 2))
    mean_b = mean_l.astype(jnp.bfloat16)

    out = pl.pallas_call(
        _fwd_kernel,
        out_shape=jax.ShapeDtypeStruct((npad, W, C2 * HP), jnp.bfloat16),
        grid=(nblocks,),
        in_specs=[
            pl.BlockSpec((nb, W, C0 * H), lambda n: (n, 0, 0)),
            pl.BlockSpec((1, 1, C0 * H), lambda n: (0, 0, 0)),
            pl.BlockSpec((C0 * H, 3 * C1 * H), lambda n: (0, 0)),
            pl.BlockSpec((1, C1 * H), lambda n: (0, 0)),
            pl.BlockSpec((C1 * H, 3 * C1 * H), lambda n: (0, 0)),
            pl.BlockSpec((1, C1 * H), lambda n: (0, 0)),
            pl.BlockSpec((C1 * H, 3 * C2 * HP), lambda n: (0, 0)),
            pl.BlockSpec((1, C2 * HP), lambda n: (0, 0)),
        ],
        out_specs=pl.BlockSpec((nb, W, C2 * HP), lambda n: (n, 0, 0)),
        compiler_params=pltpu.CompilerParams(
            dimension_semantics=("parallel",)),
    )(xt, mean_b, w1, b1, w2, b2, w3, b3)

    sel = jnp.eye(W, dtype=jnp.bfloat16)[::2, :]
    ep = jnp.einsum("nvch,wv->nchw", out[:N].reshape(N, W, C2, HP), sel)
    return ep.astype(x_nchw.dtype)


# speedup vs baseline: 2.4517x; 1.2264x over previous
import jax
import jax.numpy as jnp
from jax.experimental import pallas as pl
from jax.experimental.pallas import tpu as pltpu


H = W = 16
C0, C1, C2 = 3, 8, 16
HP, WP = H // 2, W // 2
_NB = 1024


def _extract_taps(t, width):
    cin = t.shape[1] // width
    cout = t.shape[2] // width
    tr = t.reshape(3, cin, width, cout, width)
    return jnp.stack([tr[:, :, dj, :, 1] for dj in range(3)], axis=1)


def _extract_taps_folded(t3f):
    tr = t3f.reshape(3, C1, W, C2, WP)
    return jnp.stack([tr[:, :, 2 * dj, :, 1] for dj in range(3)], axis=1)


def _toeplitz(w, width):
    kh, kw, cin, cout = w.shape
    wi = jnp.arange(width)[:, None]
    wo = jnp.arange(width)[None, :]
    dj = wi - wo + 1
    valid = ((dj >= 0) & (dj < kw)).astype(w.dtype)
    djc = jnp.clip(dj, 0, kw - 1)
    mats = []
    for di in range(kh):
        blk = w[di][djc] * valid[:, :, None, None]
        blk = jnp.transpose(blk, (2, 0, 3, 1))
        mats.append(blk.reshape(cin * width, cout * width))
    return jnp.stack(mats, axis=0)


def _swapped_weights(t1, t2, t3):
    w1 = jnp.transpose(_extract_taps(t1, W), (1, 0, 2, 3))
    w2 = jnp.transpose(_extract_taps(t2, W), (1, 0, 2, 3))
    w3 = jnp.transpose(_extract_taps_folded(t3), (1, 0, 2, 3))
    t1s = _toeplitz(w1, H)
    t2s = _toeplitz(w2, H)
    t3h = _toeplitz(w3, HP)
    rows = jnp.arange(C1 * HP)
    src = (rows // HP) * H + (rows % HP) * 2
    sel = jnp.zeros((C1 * H, C1 * HP), t3h.dtype).at[src, rows].set(1.0)
    t3s = jnp.einsum("sk,dko->dso", sel, t3h)
    cat = lambda t: jnp.concatenate([t[0], t[1], t[2]], axis=1)
    return (cat(t1s).astype(jnp.bfloat16), cat(t2s).astype(jnp.bfloat16),
            cat(t3s).astype(jnp.bfloat16))


def _conv_block(a, w_ref, b_ref, relu, shift=1, f32_tail=False):
    nb, rows, kin = a.shape
    kout = w_ref.shape[1] // 3
    z = jnp.dot(a.reshape(nb * rows, kin), w_ref[...],
                preferred_element_type=jnp.float32)
    if not f32_tail:
        z = z.astype(jnp.bfloat16)
    z = z.reshape(nb, rows, 3 * kout)
    z0 = z[..., :kout]
    z1 = z[..., kout:2 * kout]
    z2 = z[..., 2 * kout:]
    zero = jnp.zeros((nb, shift, kout), z.dtype)
    acc = (z1
           + jnp.concatenate([zero, z0[:, :rows - shift]], axis=1)
           + jnp.concatenate([z2[:, shift:], zero], axis=1)
           + b_ref[...].astype(z.dtype))
    if relu:
        acc = jnp.maximum(acc, 0.0)
    return acc


def _fwd_kernel(x_ref, mean_ref, w1_ref, b1_ref, w2_ref, b2_ref,
                w3_ref, b3_ref, o_ref):
    nb = x_ref.shape[0]
    lhs = x_ref[...] - mean_ref[...]

    a1 = _conv_block(lhs, w1_ref, b1_ref, relu=True,
                     f32_tail=True).astype(jnp.bfloat16)
    a2 = _conv_block(a1, w2_ref, b2_ref, relu=True,
                     f32_tail=True).astype(jnp.bfloat16)

    w32 = pltpu.bitcast(a2.reshape(nb * W, C1 * W), jnp.int32)
    lo = pltpu.bitcast(jax.lax.shift_left(w32, 16), jnp.float32)
    hi = pltpu.bitcast(jax.lax.bitwise_and(
        w32, jnp.int32(-65536)), jnp.float32)
    m = jnp.maximum(lo, hi).reshape(nb, WP, C1 * W)
    m_shift = jnp.concatenate([m[..., 1:], m[..., :1]], axis=-1)
    mw = jnp.maximum(m, m_shift).astype(jnp.bfloat16)

    a3 = _conv_block(mw, w3_ref, b3_ref, relu=False,
                     f32_tail=True)
    o_ref[...] = a3.astype(o_ref.dtype)


def kernel(x_nchw, mean_l, t1, b1, t2, b2, t3, b3):
    N = x_nchw.shape[0]
    nb = min(_NB, N)
    nblocks = pl.cdiv(N, nb)
    npad = nblocks * nb
    if npad != N:
        x_nchw = jnp.pad(x_nchw, ((0, npad - N), (0, 0), (0, 0), (0, 0)))

    w1, w2, w3 = _swapped_weights(t1, t2, t3)

    xt = jax.lax.reshape(x_nchw.astype(jnp.bfloat16), (npad, W, C0 * H),
                         dimensions=(0, 3, 1, 2))
    mean_b = mean_l.astype(jnp.bfloat16)

    out = pl.pallas_call(
        _fwd_kernel,
        out_shape=jax.ShapeDtypeStruct((npad, WP, C2 * HP), jnp.bfloat16),
        grid=(nblocks,),
        in_specs=[
            pl.BlockSpec((nb, W, C0 * H), lambda n: (n, 0, 0)),
            pl.BlockSpec((1, 1, C0 * H), lambda n: (0, 0, 0)),
            pl.BlockSpec((C0 * H, 3 * C1 * H), lambda n: (0, 0)),
            pl.BlockSpec((1, C1 * H), lambda n: (0, 0)),
            pl.BlockSpec((C1 * H, 3 * C1 * H), lambda n: (0, 0)),
            pl.BlockSpec((1, C1 * H), lambda n: (0, 0)),
            pl.BlockSpec((C1 * H, 3 * C2 * HP), lambda n: (0, 0)),
            pl.BlockSpec((1, C2 * HP), lambda n: (0, 0)),
        ],
        out_specs=pl.BlockSpec((nb, WP, C2 * HP), lambda n: (n, 0, 0)),
        compiler_params=pltpu.CompilerParams(
            dimension_semantics=("parallel",)),
    )(xt, mean_b, w1, b1, w2, b2, w3, b3)

    sel = jnp.eye(WP, dtype=jnp.bfloat16)
    ep = jnp.einsum("nvch,wv->nchw", out[:N].reshape(N, WP, C2, HP), sel)
    return ep.astype(x_nchw.dtype)


# speedup vs baseline: 2.6722x; 1.0899x over previous
import jax
import jax.numpy as jnp
from jax.experimental import pallas as pl
from jax.experimental.pallas import tpu as pltpu


H = W = 16
C0, C1, C2 = 3, 8, 16
HP, WP = H // 2, W // 2
_NB = 1024


def _extract_taps(t, width):
    cin = t.shape[1] // width
    cout = t.shape[2] // width
    tr = t.reshape(3, cin, width, cout, width)
    return jnp.stack([tr[:, :, dj, :, 1] for dj in range(3)], axis=1)


def _extract_taps_folded(t3f):
    tr = t3f.reshape(3, C1, W, C2, WP)
    return jnp.stack([tr[:, :, 2 * dj, :, 1] for dj in range(3)], axis=1)


def _toeplitz(w, width):
    kh, kw, cin, cout = w.shape
    wi = jnp.arange(width)[:, None]
    wo = jnp.arange(width)[None, :]
    dj = wi - wo + 1
    valid = ((dj >= 0) & (dj < kw)).astype(w.dtype)
    djc = jnp.clip(dj, 0, kw - 1)
    mats = []
    for di in range(kh):
        blk = w[di][djc] * valid[:, :, None, None]
        blk = jnp.transpose(blk, (2, 0, 3, 1))
        mats.append(blk.reshape(cin * width, cout * width))
    return jnp.stack(mats, axis=0)


def _swapped_weights(t1, t2, t3):
    w1 = jnp.transpose(_extract_taps(t1, W), (1, 0, 2, 3))
    w2 = jnp.transpose(_extract_taps(t2, W), (1, 0, 2, 3))
    w3 = jnp.transpose(_extract_taps_folded(t3), (1, 0, 2, 3))
    t1s = _toeplitz(w1, H)
    t2s = _toeplitz(w2, H)
    t3h = _toeplitz(w3, HP)
    rows = jnp.arange(C1 * HP)
    src = (rows // HP) * H + (rows % HP) * 2
    sel = jnp.zeros((C1 * H, C1 * HP), t3h.dtype).at[src, rows].set(1.0)
    t3s = jnp.einsum("sk,dko->dso", sel, t3h)
    cat = lambda t: jnp.concatenate([t[0], t[1], t[2]], axis=1)
    return (cat(t1s).astype(jnp.bfloat16), cat(t2s).astype(jnp.bfloat16),
            cat(t3s).astype(jnp.bfloat16))


def _conv_block(a, w_ref, b_ref, relu, shift=1, f32_tail=False):
    nb, rows, kin = a.shape
    kout = w_ref.shape[1] // 3
    z = jnp.dot(a.reshape(nb * rows, kin), w_ref[...],
                preferred_element_type=jnp.float32)
    if not f32_tail:
        z = z.astype(jnp.bfloat16)
    z = z.reshape(nb, rows, 3 * kout)
    z0 = z[..., :kout]
    z1 = z[..., kout:2 * kout]
    z2 = z[..., 2 * kout:]
    zero = jnp.zeros((nb, shift, kout), z.dtype)
    acc = (z1
           + jnp.concatenate([zero, z0[:, :rows - shift]], axis=1)
           + jnp.concatenate([z2[:, shift:], zero], axis=1)
           + b_ref[...].astype(z.dtype))
    if relu:
        acc = jnp.maximum(acc, 0.0)
    return acc


def _fwd_kernel(x_ref, mean_ref, w1_ref, b1_ref, w2_ref, b2_ref,
                w3_ref, b3_ref, o_ref):
    nb = x_ref.shape[0]
    x = x_ref[...]
    lhs = jnp.concatenate([x[:, c] for c in range(C0)], axis=-1)
    lhs = lhs - mean_ref[...]

    a1 = _conv_block(lhs, w1_ref, b1_ref, relu=True,
                     f32_tail=True).astype(jnp.bfloat16)
    a2 = _conv_block(a1, w2_ref, b2_ref, relu=True,
                     f32_tail=True).astype(jnp.bfloat16)

    w32 = pltpu.bitcast(a2.reshape(nb * W, C1 * W), jnp.int32)
    lo = pltpu.bitcast(jax.lax.shift_left(w32, 16), jnp.float32)
    hi = pltpu.bitcast(jax.lax.bitwise_and(
        w32, jnp.int32(-65536)), jnp.float32)
    m = jnp.maximum(lo, hi).reshape(nb, WP, C1 * W)
    m_shift = jnp.concatenate([m[..., 1:], m[..., :1]], axis=-1)
    mw = jnp.maximum(m, m_shift).astype(jnp.bfloat16)

    a3 = _conv_block(mw, w3_ref, b3_ref, relu=False,
                     f32_tail=True)
    o_ref[...] = a3.astype(o_ref.dtype)


def kernel(x_nchw, mean_l, t1, b1, t2, b2, t3, b3):
    N = x_nchw.shape[0]
    nb = min(_NB, N)
    nblocks = pl.cdiv(N, nb)
    npad = nblocks * nb
    if npad != N:
        x_nchw = jnp.pad(x_nchw, ((0, npad - N), (0, 0), (0, 0), (0, 0)))

    w1, w2, w3 = _swapped_weights(t1, t2, t3)

    eye_h = jnp.eye(H, dtype=jnp.bfloat16)
    xt = jnp.einsum("nchw,uh->ncwu", x_nchw.astype(jnp.bfloat16), eye_h)
    mean_b = mean_l.astype(jnp.bfloat16)

    out = pl.pallas_call(
        _fwd_kernel,
        out_shape=jax.ShapeDtypeStruct((npad, WP, C2 * HP), jnp.bfloat16),
        grid=(nblocks,),
        in_specs=[
            pl.BlockSpec((nb, C0, W, H), lambda n: (n, 0, 0, 0)),
            pl.BlockSpec((1, 1, C0 * H), lambda n: (0, 0, 0)),
            pl.BlockSpec((C0 * H, 3 * C1 * H), lambda n: (0, 0)),
            pl.BlockSpec((1, C1 * H), lambda n: (0, 0)),
            pl.BlockSpec((C1 * H, 3 * C1 * H), lambda n: (0, 0)),
            pl.BlockSpec((1, C1 * H), lambda n: (0, 0)),
            pl.BlockSpec((C1 * H, 3 * C2 * HP), lambda n: (0, 0)),
            pl.BlockSpec((1, C2 * HP), lambda n: (0, 0)),
        ],
        out_specs=pl.BlockSpec((nb, WP, C2 * HP), lambda n: (n, 0, 0)),
        compiler_params=pltpu.CompilerParams(
            dimension_semantics=("parallel",)),
    )(xt, mean_b, w1, b1, w2, b2, w3, b3)

    sel = jnp.eye(WP, dtype=jnp.bfloat16)
    ep = jnp.einsum("nvch,wv->nchw", out[:N].reshape(N, WP, C2, HP), sel)
    return ep.astype(x_nchw.dtype)
